# baseline (speedup 1.0000x reference)
import jax
import jax.numpy as jnp
from jax.experimental import pallas as pl
from jax.experimental.pallas import tpu as pltpu
from jax.experimental.pallas import tpu_sc as plsc

DIM = 256
TM = 512
TN = 512


WINDOW = 2048


def _argmin_body(x_ref, et_ref, ind_ref, best_ref, bestn_ref):
    n_tiles = et_ref.shape[0]
    tiles_per_win = WINDOW // TN
    n_windows = n_tiles // tiles_per_win
    x = x_ref[...]
    xb = x.astype(jnp.bfloat16)
    x2 = jnp.sum(x * x, axis=1, keepdims=True)

    def win_body(w, carry):
        m, idx = carry
        best_ref[...] = jnp.full((TM, TN), -jnp.inf, jnp.float32)
        bestn_ref[...] = jnp.zeros((TM, TN), jnp.int32)

        def body(t, c):
            n = w * tiles_per_win + t
            et = et_ref[n]
            s = jax.lax.dot_general(
                xb, et.astype(jnp.bfloat16),
                (((1,), (0,)), ((), ())),
                preferred_element_type=jnp.float32)
            e2 = jnp.sum(et * et, axis=0, keepdims=True)
            score = -((x2 - (s + s)) + e2)
            b = best_ref[...]
            upd = score > b
            best_ref[...] = jnp.where(upd, score, b)
            nb = bestn_ref[...]
            bestn_ref[...] = jnp.where(upd, jnp.full_like(nb, n), nb)
            return c

        jax.lax.fori_loop(0, tiles_per_win, body, 0)

        best = best_ref[...]
        wmax = jnp.max(best, axis=1, keepdims=True)
        lane = jax.lax.broadcasted_iota(jnp.int32, (TM, TN), 1)
        idxv = bestn_ref[...] * TN + lane
        masked = jnp.where(best == wmax, idxv, jnp.int32(2**30))
        widx = jnp.min(masked, axis=1, keepdims=True)

        upd = wmax > m
        idx = jnp.where(upd, widx, idx)
        m = jnp.where(upd, wmax.astype(jnp.bfloat16).astype(jnp.float32), m)
        return (m, idx)

    m0 = jnp.full((TM, 1), -jnp.inf, jnp.float32)
    i0 = jnp.zeros((TM, 1), jnp.int32)
    _, idx = jax.lax.fori_loop(0, n_windows, win_body, (m0, i0))
    ind_ref[...] = idx.reshape(1, TM, 1)


def _argmin_call(xf, et3, interpret=False):
    ntok = xf.shape[0]
    n_tiles = et3.shape[0]
    return pl.pallas_call(
        _argmin_body,
        grid=(ntok // TM,),
        in_specs=[
            pl.BlockSpec((TM, DIM), lambda i: (i, 0)),
            pl.BlockSpec((n_tiles, DIM, TN), lambda i: (0, 0, 0)),
        ],
        out_specs=pl.BlockSpec((1, TM, 1), lambda i: (i, 0, 0)),
        out_shape=jax.ShapeDtypeStruct((ntok // TM, TM, 1), jnp.int32),
        scratch_shapes=[
            pltpu.VMEM((TM, TN), jnp.float32),
            pltpu.VMEM((TM, TN), jnp.int32),
        ],
        compiler_params=pltpu.CompilerParams(
            dimension_semantics=("parallel",)),
        interpret=interpret,
    )(xf, et3)


def _sc_gather(embed, idx):
    ntok = idx.shape[1]
    window = 128
    mesh = plsc.VectorSubcoreMesh(
        core_axis_name="core", subcore_axis_name="subcore")

    @pl.kernel(out_type=jax.ShapeDtypeStruct((ntok, embed.shape[1]),
                                             embed.dtype),
               mesh=mesh)
    def k(x_hbm, i_hbm, o_hbm):
        def body(i_vmem, o_vmem):
            pltpu.sync_copy(x_hbm.at[i_vmem.at[0]], o_vmem)

        pltpu.emit_pipeline(
            body,
            grid=(ntok // window,),
            in_specs=[pl.BlockSpec((1, window), index_map=lambda i: (0, i))],
            out_specs=[pl.BlockSpec((window, embed.shape[1]),
                                    index_map=lambda i: (i, 0))],
            core_axis_name=("core", "subcore"),
            dimension_semantics=(pltpu.PARALLEL,),
        )(i_hbm, o_hbm)

    return k(embed, idx)


def kernel(x, embed):
    shape = x.shape
    xf = x.reshape(-1, shape[-1])
    ntok = xf.shape[0]
    ncode = embed.shape[0]
    et3 = jnp.transpose(embed.T.reshape(DIM, ncode // TN, TN), (1, 0, 2))
    ind3 = _argmin_call(xf, et3)
    idx = ind3.reshape(1, ntok)
    quant = _sc_gather(embed, idx)
    embed_ind = idx.reshape(shape[:-1])
    quantize = quant.reshape(shape)
    return (quantize, embed_ind)


# speedup vs baseline: 1.2536x; 1.2536x over previous
import jax
import jax.numpy as jnp
from jax.experimental import pallas as pl
from jax.experimental.pallas import tpu as pltpu
from jax.experimental.pallas import tpu_sc as plsc

DIM = 256
TM = 512
WINDOW = 2048


def _argmin_body(x_ref, et_ref, ind_ref):
    n_windows = et_ref.shape[0]
    x = x_ref[...]
    xb = x.astype(jnp.bfloat16)
    x2 = jnp.sum(x * x, axis=1, keepdims=True)

    def win_body(w, carry):
        m, idx = carry
        et = et_ref[w]
        s = jax.lax.dot_general(
            xb, et.astype(jnp.bfloat16),
            (((1,), (0,)), ((), ())),
            preferred_element_type=jnp.float32)
        e2 = jnp.sum(et * et, axis=0, keepdims=True)
        score = -((x2 - (s + s)) + e2)

        wmax = jnp.max(score, axis=1, keepdims=True)
        lane = jax.lax.broadcasted_iota(jnp.int32, (TM, WINDOW), 1)
        masked = jnp.where(score == wmax, lane, jnp.int32(2**30))
        widx = jnp.min(masked, axis=1, keepdims=True) + w * WINDOW

        upd = wmax > m
        idx = jnp.where(upd, widx, idx)
        m = jnp.where(upd, wmax.astype(jnp.bfloat16).astype(jnp.float32), m)
        return (m, idx)

    m0 = jnp.full((TM, 1), -jnp.inf, jnp.float32)
    i0 = jnp.zeros((TM, 1), jnp.int32)
    _, idx = jax.lax.fori_loop(0, n_windows, win_body, (m0, i0))
    ind_ref[...] = idx.reshape(1, TM, 1)


def _argmin_call(xf, et3, interpret=False):
    ntok = xf.shape[0]
    n_windows = et3.shape[0]
    return pl.pallas_call(
        _argmin_body,
        grid=(ntok // TM,),
        in_specs=[
            pl.BlockSpec((TM, DIM), lambda i: (i, 0)),
            pl.BlockSpec((n_windows, DIM, WINDOW), lambda i: (0, 0, 0)),
        ],
        out_specs=pl.BlockSpec((1, TM, 1), lambda i: (i, 0, 0)),
        out_shape=jax.ShapeDtypeStruct((ntok // TM, TM, 1), jnp.int32),
        compiler_params=pltpu.CompilerParams(
            dimension_semantics=("parallel",)),
        interpret=interpret,
    )(xf, et3)


def _sc_gather(embed, idx):
    ntok = idx.shape[1]
    window = 128
    mesh = plsc.VectorSubcoreMesh(
        core_axis_name="core", subcore_axis_name="subcore")

    @pl.kernel(out_type=jax.ShapeDtypeStruct((ntok, embed.shape[1]),
                                             embed.dtype),
               mesh=mesh)
    def k(x_hbm, i_hbm, o_hbm):
        def body(i_vmem, o_vmem):
            pltpu.sync_copy(x_hbm.at[i_vmem.at[0]], o_vmem)

        pltpu.emit_pipeline(
            body,
            grid=(ntok // window,),
            in_specs=[pl.BlockSpec((1, window), index_map=lambda i: (0, i))],
            out_specs=[pl.BlockSpec((window, embed.shape[1]),
                                    index_map=lambda i: (i, 0))],
            core_axis_name=("core", "subcore"),
            dimension_semantics=(pltpu.PARALLEL,),
        )(i_hbm, o_hbm)

    return k(embed, idx)


def kernel(x, embed):
    shape = x.shape
    xf = x.reshape(-1, shape[-1])
    ntok = xf.shape[0]
    ncode = embed.shape[0]
    et3 = jnp.transpose(embed.T.reshape(DIM, ncode // WINDOW, WINDOW),
                        (1, 0, 2))
    ind3 = _argmin_call(xf, et3)
    idx = ind3.reshape(1, ntok)
    quant = _sc_gather(embed, idx)
    embed_ind = idx.reshape(shape[:-1])
    quantize = quant.reshape(shape)
    return (quantize, embed_ind)


# speedup vs baseline: 1.4258x; 1.1374x over previous
import jax
import jax.numpy as jnp
from jax.experimental import pallas as pl
from jax.experimental.pallas import tpu as pltpu
from jax.experimental.pallas import tpu_sc as plsc

DIM = 256
TM = 512
WINDOW = 2048


def _argmin_body(x_ref, et_ref, ind_ref):
    n_windows = et_ref.shape[0]
    x = x_ref[...]
    xb = x.astype(jnp.bfloat16)
    x2 = jnp.sum(x * x, axis=1, keepdims=True)

    def win_body(w, carry):
        m, idx = carry
        et = et_ref[w]
        s2 = jax.lax.dot_general(
            xb, et.astype(jnp.bfloat16),
            (((1,), (0,)), ((), ())),
            preferred_element_type=jnp.float32)
        e2 = jnp.sum(et * et, axis=0, keepdims=True) * 0.25
        d = (x2 - s2) + e2

        wmin = jnp.min(d, axis=1, keepdims=True)
        lane = jax.lax.broadcasted_iota(jnp.int32, (TM, WINDOW), 1)
        masked = jnp.where(d == wmin, lane, jnp.int32(2**30))
        widx = jnp.min(masked, axis=1, keepdims=True) + w * WINDOW

        upd = wmin < m
        idx = jnp.where(upd, widx, idx)
        m = jnp.where(upd, wmin.astype(jnp.bfloat16).astype(jnp.float32), m)
        return (m, idx)

    m0 = jnp.full((TM, 1), jnp.inf, jnp.float32)
    i0 = jnp.zeros((TM, 1), jnp.int32)
    _, idx = jax.lax.fori_loop(0, n_windows, win_body, (m0, i0))
    ind_ref[...] = idx.reshape(1, TM, 1)


def _argmin_call(xf, et3, interpret=False):
    ntok = xf.shape[0]
    n_windows = et3.shape[0]
    return pl.pallas_call(
        _argmin_body,
        grid=(ntok // TM,),
        in_specs=[
            pl.BlockSpec((TM, DIM), lambda i: (i, 0)),
            pl.BlockSpec((n_windows, DIM, WINDOW), lambda i: (0, 0, 0)),
        ],
        out_specs=pl.BlockSpec((1, TM, 1), lambda i: (i, 0, 0)),
        out_shape=jax.ShapeDtypeStruct((ntok // TM, TM, 1), jnp.int32),
        compiler_params=pltpu.CompilerParams(
            dimension_semantics=("parallel",)),
        interpret=interpret,
    )(xf, et3)


def _sc_gather(embed, idx):
    ntok = idx.shape[1]
    window = 128
    mesh = plsc.VectorSubcoreMesh(
        core_axis_name="core", subcore_axis_name="subcore")

    @pl.kernel(out_type=jax.ShapeDtypeStruct((ntok, embed.shape[1]),
                                             embed.dtype),
               mesh=mesh)
    def k(x_hbm, i_hbm, o_hbm):
        def body(i_vmem, o_vmem):
            pltpu.sync_copy(x_hbm.at[i_vmem.at[0]], o_vmem)

        pltpu.emit_pipeline(
            body,
            grid=(ntok // window,),
            in_specs=[pl.BlockSpec((1, window), index_map=lambda i: (0, i))],
            out_specs=[pl.BlockSpec((window, embed.shape[1]),
                                    index_map=lambda i: (i, 0))],
            core_axis_name=("core", "subcore"),
            dimension_semantics=(pltpu.PARALLEL,),
        )(i_hbm, o_hbm)

    return k(embed, idx)


def kernel(x, embed):
    shape = x.shape
    xf = x.reshape(-1, shape[-1])
    ntok = xf.shape[0]
    ncode = embed.shape[0]
    et3 = jnp.transpose((2.0 * embed).T.reshape(DIM, ncode // WINDOW, WINDOW),
                        (1, 0, 2))
    ind3 = _argmin_call(xf, et3)
    idx = ind3.reshape(1, ntok)
    quant = _sc_gather(embed, idx)
    embed_ind = idx.reshape(shape[:-1])
    quantize = quant.reshape(shape)
    return (quantize, embed_ind)
